# Initial kernel scaffold; baseline (speedup 1.0000x reference)
import jax
import jax.numpy as jnp
from jax.experimental import pallas as pl


def kernel(x, v, edge_index, edge_rots, edge_attr, ptr, params):
    raise NotImplementedError("write your pallas kernel here")


# speedup vs baseline: 3.7740x; 3.7740x over previous
import functools

import jax
import jax.numpy as jnp
from jax import lax
from jax.experimental import pallas as pl
from jax.experimental.pallas import tpu as pltpu
from jax.experimental.pallas import tpu_sc as plsc

F32 = jnp.float32
EMB = 128
MD = 12
NW = 32
CH = 512
SEG = CH // 128
BN = 1024
BE = 2560


def _node_block(i):
    return (i, 0)


def _full_block(i):
    return (0, 0)


def _prologue_body(xr, gr, br, wtr, pbr, out):
    xx = xr[...]
    mu = jnp.mean(xx, axis=-1, keepdims=True)
    var = jnp.mean((xx - mu) ** 2, axis=-1, keepdims=True)
    xn = (xx - mu) * lax.rsqrt(var + 1e-5) * gr[...] + br[...]
    out[...] = jnp.dot(xn, wtr[...], preferred_element_type=F32) + pbr[...]


def _gh_body(hr, vr, wghr, wgvr, whr, gout, hout):
    hh = hr[...]
    vv = vr[...]
    gout[...] = (jnp.dot(hh, wghr[...], preferred_element_type=F32)
                 + jnp.dot(vv, wgvr[...], preferred_element_type=F32))
    hout[...] = jnp.dot(hh, whr[...], preferred_element_type=F32)


def _mlp_body(zar, zbr, vjr, rotr, attrr, wdr, wear, b1r, w2r, b2r, out):
    z = zar[...] + zbr[...] + b1r[...]
    V = vjr[...]
    Rr = rotr[...]
    wd = wdr[...]
    for i in range(3):
        acc = (Rr[:, 3 * i:3 * i + 1] * V[:, 0:4]
               + Rr[:, 3 * i + 1:3 * i + 2] * V[:, 4:8]
               + Rr[:, 3 * i + 2:3 * i + 3] * V[:, 8:12])
        z = z + jnp.dot(acc, wd[4 * i:4 * i + 4, :], preferred_element_type=F32)
    z = z + jnp.dot(attrr[...], wear[...], preferred_element_type=F32)
    u = jax.nn.gelu(z)
    h2 = jax.nn.gelu(jnp.dot(u, w2r[...], preferred_element_type=F32) + b2r[...])
    out[...] = h2


def _node_body(hr, vr, s0r, s1r, c0r, c1r, w3r, b3r, gr, br, wvhr, wvvr,
               hout, vout):
    S = s0r[...] + s1r[...]
    c = c0r[:, 0:1] + c1r[:, 0:1]
    cmax = jnp.maximum(c, 1.0)
    ind = jnp.minimum(c, 1.0)
    upd = jnp.dot(S, w3r[...], preferred_element_type=F32) / cmax + b3r[...] * ind
    t = hr[...] + upd
    mu = jnp.mean(t, axis=-1, keepdims=True)
    var = jnp.mean((t - mu) ** 2, axis=-1, keepdims=True)
    hn = (t - mu) * lax.rsqrt(var + 1e-5) * gr[...] + br[...]
    hout[...] = hn
    vout[...] = (vr[...] + jnp.dot(hn, wvhr[...], preferred_element_type=F32)
                 + jnp.dot(vr[...], wvvr[...], preferred_element_type=F32))


def _make_gather(npad, e):
    nch = e // CH
    iters = (nch + NW - 1) // NW
    mesh = plsc.VectorSubcoreMesh(core_axis_name="c", subcore_axis_name="s")

    @functools.partial(
        pl.kernel, mesh=mesh,
        out_type=(jax.ShapeDtypeStruct((e, EMB), F32),
                  jax.ShapeDtypeStruct((e, EMB), F32),
                  jax.ShapeDtypeStruct((e, EMB), F32)),
        scratch_types=[
            pltpu.VMEM((SEG, 128), jnp.int32),
            pltpu.VMEM((SEG, 128), jnp.int32),
            pltpu.VMEM((CH, EMB), F32),
            pltpu.SemaphoreType.DMA,
        ],
    )
    def gather_k(g_hbm, h_hbm, vt_hbm, row_hbm, col_hbm, za_hbm, zb_hbm,
                 vj_hbm, idxr, idxc, buf, sem):
        wid = lax.axis_index("s") * 2 + lax.axis_index("c")

        def chunk(k, carry):
            ci = wid + NW * k

            @pl.when(ci < nch)
            def _():
                pltpu.sync_copy(row_hbm.at[pl.ds(ci * SEG, SEG)], idxr)
                pltpu.sync_copy(col_hbm.at[pl.ds(ci * SEG, SEG)], idxc)
                cps = [pltpu.async_copy(g_hbm.at[idxr.at[j]],
                                        buf.at[pl.ds(j * 128, 128)], sem)
                       for j in range(SEG)]
                for cp in cps:
                    cp.wait()
                pltpu.sync_copy(buf, za_hbm.at[pl.ds(ci * CH, CH)])
                cps = [pltpu.async_copy(h_hbm.at[idxc.at[j]],
                                        buf.at[pl.ds(j * 128, 128)], sem)
                       for j in range(SEG)]
                for cp in cps:
                    cp.wait()
                pltpu.sync_copy(buf, zb_hbm.at[pl.ds(ci * CH, CH)])
                cps = [pltpu.async_copy(vt_hbm.at[idxc.at[j]],
                                        buf.at[pl.ds(j * 128, 128)], sem)
                       for j in range(SEG)]
                for cp in cps:
                    cp.wait()
                pltpu.sync_copy(buf, vj_hbm.at[pl.ds(ci * CH, CH)])

            return carry

        lax.fori_loop(0, iters, chunk, 0)

    return gather_k


def _make_scatter(npad, e):
    chs = 256
    seg_s = chs // 128
    nch = e // chs
    iters = (nch + NW - 1) // NW
    rows_per_tile = npad // 16
    nzc = rows_per_tile // 128
    mesh = plsc.VectorSubcoreMesh(core_axis_name="c", subcore_axis_name="s")

    @functools.partial(
        pl.kernel, mesh=mesh,
        out_type=jax.ShapeDtypeStruct((2, npad, EMB), F32),
        scratch_types=[
            pltpu.VMEM((seg_s, 128), jnp.int32),
            pltpu.VMEM((chs, EMB), F32),
            pltpu.VMEM_SHARED((npad, EMB), F32),
        ],
    )
    def scatter_k(h2_hbm, row_hbm, zeros_hbm, s_hbm, idxr, buf, acc):
        cid = lax.axis_index("c")
        sid = lax.axis_index("s")
        wid = sid * 2 + cid
        base = sid * rows_per_tile
        for kk in range(nzc):
            pltpu.sync_copy(zeros_hbm, acc.at[pl.ds(base + kk * 128, 128)])
        plsc.subcore_barrier()

        def chunk(k, carry):
            ci = wid + NW * k

            @pl.when(ci < nch)
            def _():
                pltpu.sync_copy(row_hbm.at[pl.ds(ci * seg_s, seg_s)], idxr)
                pltpu.sync_copy(h2_hbm.at[pl.ds(ci * chs, chs)], buf)
                for j in range(seg_s):
                    pltpu.sync_copy(buf.at[pl.ds(j * 128, 128)],
                                    acc.at[idxr.at[j]], add=True)

            return carry

        lax.fori_loop(0, iters, chunk, 0)
        plsc.subcore_barrier()
        for kk in range(nzc):
            sl = pl.ds(base + kk * 128, 128)
            pltpu.sync_copy(acc.at[sl], s_hbm.at[cid, sl])

    return scatter_k


def _make_counts(npad, e):
    nch = e // CH
    iters = (nch + NW - 1) // NW
    rows_per_tile = npad // 16
    nzc = rows_per_tile // 128
    mesh = plsc.VectorSubcoreMesh(core_axis_name="c", subcore_axis_name="s")

    @functools.partial(
        pl.kernel, mesh=mesh,
        out_type=jax.ShapeDtypeStruct((2, npad, EMB), F32),
        scratch_types=[
            pltpu.VMEM((SEG, 128), jnp.int32),
            pltpu.VMEM((128, EMB), F32),
            pltpu.VMEM_SHARED((npad, EMB), F32),
        ],
    )
    def counts_k(row_hbm, ones_hbm, zeros_hbm, c_hbm, idxr, ones_v, acc):
        cid = lax.axis_index("c")
        sid = lax.axis_index("s")
        wid = sid * 2 + cid
        base = sid * rows_per_tile
        pltpu.sync_copy(ones_hbm, ones_v)
        for kk in range(nzc):
            pltpu.sync_copy(zeros_hbm, acc.at[pl.ds(base + kk * 128, 128)])
        plsc.subcore_barrier()

        def chunk(k, carry):
            ci = wid + NW * k

            @pl.when(ci < nch)
            def _():
                pltpu.sync_copy(row_hbm.at[pl.ds(ci * SEG, SEG)], idxr)
                for j in range(SEG):
                    pltpu.sync_copy(ones_v, acc.at[idxr.at[j]], add=True)

            return carry

        lax.fori_loop(0, iters, chunk, 0)
        plsc.subcore_barrier()
        for kk in range(nzc):
            sl = pl.ds(base + kk * 128, 128)
            pltpu.sync_copy(acc.at[sl], c_hbm.at[cid, sl])

    return counts_k


def kernel(x, v, edge_index, edge_rots, edge_attr, ptr, params):
    n, in_dim = x.shape
    e = edge_index.shape[1]
    npad = ((n + BN - 1) // BN) * BN

    row = edge_index[0].astype(jnp.int32)
    col = edge_index[1].astype(jnp.int32)
    row2d = row.reshape(e // 128, 128)
    col2d = col.reshape(e // 128, 128)
    rot16 = jnp.pad(edge_rots.reshape(e, 9), ((0, 0), (0, 7)))
    xp = jnp.pad(x, ((0, npad - n), (0, 0)))
    v16 = jnp.pad(v, ((0, npad - n), (0, 4)))

    ones128 = jnp.ones((128, EMB), F32)
    zeros128 = jnp.zeros((128, EMB), F32)

    rs = lambda a: a.reshape(1, -1)

    grid_n = npad // BN
    node_sp = lambda w: pl.BlockSpec((BN, w), _node_block)
    edge_sp = lambda w: pl.BlockSpec((BE, w), _node_block)
    full_sp = lambda s: pl.BlockSpec(s, _full_block)

    h = pl.pallas_call(
        _prologue_body,
        grid=(grid_n,),
        in_specs=[node_sp(in_dim), full_sp((1, in_dim)), full_sp((1, in_dim)),
                  full_sp((in_dim, EMB)), full_sp((1, EMB))],
        out_specs=node_sp(EMB),
        out_shape=jax.ShapeDtypeStruct((npad, EMB), F32),
    )(xp, rs(params['in_g']), rs(params['in_b']), params['proj_W'].T,
      rs(params['proj_b']))

    counts_k = _make_counts(npad, e)
    cpair = counts_k(row2d, ones128, zeros128)
    c0, c1 = cpair[0], cpair[1]

    gather_k = _make_gather(npad, e)
    scatter_k = _make_scatter(npad, e)

    inv = jnp.array([3 * m + i for i in range(3) for m in range(4)],
                    dtype=jnp.int32)

    for p in params['layers']:
        W1 = p['W1']
        W1xi = W1[:, 0:EMB]
        W1xj = W1[:, EMB:2 * EMB]
        W1vi = W1[:, 2 * EMB:2 * EMB + MD]
        W1vr = W1[:, 2 * EMB + MD:2 * EMB + 2 * MD]
        W1s1 = W1[:, 2 * EMB + 2 * MD:2 * EMB + 3 * MD]
        W1ea = W1[:, 2 * EMB + 3 * MD:]
        wgv = jnp.pad((W1vi + W1s1).T, ((0, 4), (0, 0)))
        wd = jnp.pad((W1vr - W1s1).T[inv], ((0, 4), (0, 0)))
        wea = W1ea.T
        wvh = jnp.pad(p['Wv'][:, :EMB].T, ((0, 0), (0, 4)))
        wvv = jnp.pad(p['Wv'][:, EMB:].T, ((0, 4), (0, 4)))

        G, H = pl.pallas_call(
            _gh_body,
            grid=(grid_n,),
            in_specs=[node_sp(EMB), node_sp(16), full_sp((EMB, EMB)),
                      full_sp((16, EMB)), full_sp((EMB, EMB))],
            out_specs=(node_sp(EMB), node_sp(EMB)),
            out_shape=(jax.ShapeDtypeStruct((npad, EMB), F32),
                       jax.ShapeDtypeStruct((npad, EMB), F32)),
        )(h, v16, W1xi.T, wgv, W1xj.T)

        vt128 = jnp.pad(
            v16[:, :MD].reshape(npad, 4, 3).transpose(0, 2, 1).reshape(npad, MD),
            ((0, 0), (0, EMB - MD)))

        za, zb, vj = gather_k(G, H, vt128, row2d, col2d)

        h2 = pl.pallas_call(
            _mlp_body,
            grid=(e // BE,),
            in_specs=[edge_sp(EMB), edge_sp(EMB), edge_sp(EMB), edge_sp(16),
                      edge_sp(16), full_sp((16, EMB)), full_sp((16, EMB)),
                      full_sp((1, EMB)), full_sp((EMB, EMB)),
                      full_sp((1, EMB))],
            out_specs=edge_sp(EMB),
            out_shape=jax.ShapeDtypeStruct((e, EMB), F32),
        )(za, zb, vj, rot16, edge_attr, wd, wea, rs(p['b1']), p['W2'].T,
          rs(p['b2']))

        spair = scatter_k(h2, row2d, zeros128)
        S0, S1 = spair[0], spair[1]

        h, v16 = pl.pallas_call(
            _node_body,
            grid=(grid_n,),
            in_specs=[node_sp(EMB), node_sp(16), node_sp(EMB), node_sp(EMB),
                      node_sp(EMB), node_sp(EMB), full_sp((EMB, EMB)),
                      full_sp((1, EMB)), full_sp((1, EMB)), full_sp((1, EMB)),
                      full_sp((EMB, 16)), full_sp((16, 16))],
            out_specs=(node_sp(EMB), node_sp(16)),
            out_shape=(jax.ShapeDtypeStruct((npad, EMB), F32),
                       jax.ShapeDtypeStruct((npad, 16), F32)),
        )(h, v16, S0, S1, c0, c1, p['W3'].T, rs(p['b3']), rs(p['ln_g']),
          rs(p['ln_b']), wvh, wvv)

    return h[:n], v16[:n, :MD]


# speedup vs baseline: 4.1304x; 1.0944x over previous
import functools

import jax
import jax.numpy as jnp
from jax import lax
from jax.experimental import pallas as pl
from jax.experimental.pallas import tpu as pltpu
from jax.experimental.pallas import tpu_sc as plsc

F32 = jnp.float32
EMB = 128
MD = 12
NW = 32
CH = 512
SEG = CH // 128
BN = 1024
BE = 2560


def _node_block(i):
    return (i, 0)


def _full_block(i):
    return (0, 0)


def _prologue_body(xr, gr, br, wtr, pbr, out):
    xx = xr[...]
    mu = jnp.mean(xx, axis=-1, keepdims=True)
    var = jnp.mean((xx - mu) ** 2, axis=-1, keepdims=True)
    xn = (xx - mu) * lax.rsqrt(var + 1e-5) * gr[...] + br[...]
    out[...] = jnp.dot(xn, wtr[...], preferred_element_type=F32) + pbr[...]


def _gh_body(hr, vr, vtr, wghr, wgvr, whr, gout, hvout):
    hh = hr[...]
    vv = vr[...]
    gout[...] = (jnp.dot(hh, wghr[...], preferred_element_type=F32)
                 + jnp.dot(vv, wgvr[...], preferred_element_type=F32))
    hcol = jnp.dot(hh, whr[...], preferred_element_type=F32)
    hu = lax.bitcast_convert_type(hcol.astype(jnp.bfloat16),
                                  jnp.uint16).astype(jnp.uint32)
    vu = lax.bitcast_convert_type(vtr[...].astype(jnp.bfloat16),
                                  jnp.uint16).astype(jnp.uint32)
    hvout[...] = lax.bitcast_convert_type(hu | (vu << 16), jnp.int32)


def _mlp_body(zar, zbvr, rotr, attrr, wdr, wear, b1r, w2r, b2r, out):
    zu = lax.bitcast_convert_type(zbvr[...], jnp.uint32)
    hcol = lax.bitcast_convert_type((zu & 0xFFFF).astype(jnp.uint16),
                                    jnp.bfloat16).astype(F32)
    V = lax.bitcast_convert_type((zu[:, :16] >> 16).astype(jnp.uint16),
                                 jnp.bfloat16).astype(F32)
    z = zar[...] + hcol + b1r[...]
    Rr = rotr[...]
    wd = wdr[...]
    for i in range(3):
        acc = (Rr[:, 3 * i:3 * i + 1] * V[:, 0:4]
               + Rr[:, 3 * i + 1:3 * i + 2] * V[:, 4:8]
               + Rr[:, 3 * i + 2:3 * i + 3] * V[:, 8:12])
        z = z + jnp.dot(acc, wd[4 * i:4 * i + 4, :], preferred_element_type=F32)
    z = z + jnp.dot(attrr[...], wear[...], preferred_element_type=F32)
    u = jax.nn.gelu(z)
    h2 = jax.nn.gelu(jnp.dot(u, w2r[...], preferred_element_type=F32) + b2r[...])
    out[...] = h2


def _node_body(hr, vr, s0r, s1r, c0r, c1r, w3r, b3r, gr, br, wvhr, wvvr,
               hout, vout):
    S = s0r[...] + s1r[...]
    c = c0r[:, 0:1] + c1r[:, 0:1]
    cmax = jnp.maximum(c, 1.0)
    ind = jnp.minimum(c, 1.0)
    upd = jnp.dot(S, w3r[...], preferred_element_type=F32) / cmax + b3r[...] * ind
    t = hr[...] + upd
    mu = jnp.mean(t, axis=-1, keepdims=True)
    var = jnp.mean((t - mu) ** 2, axis=-1, keepdims=True)
    hn = (t - mu) * lax.rsqrt(var + 1e-5) * gr[...] + br[...]
    hout[...] = hn
    vout[...] = (vr[...] + jnp.dot(hn, wvhr[...], preferred_element_type=F32)
                 + jnp.dot(vr[...], wvvr[...], preferred_element_type=F32))


def _make_gather(npad, e):
    chg = 256
    seg_g = chg // 128
    nch = e // chg
    iters = (nch + NW - 1) // NW
    mesh = plsc.VectorSubcoreMesh(core_axis_name="c", subcore_axis_name="s")

    @functools.partial(
        pl.kernel, mesh=mesh,
        out_type=(jax.ShapeDtypeStruct((e, EMB), F32),
                  jax.ShapeDtypeStruct((e, EMB), jnp.int32)),
        scratch_types=[
            pltpu.VMEM((seg_g, 128), jnp.int32),
            pltpu.VMEM((seg_g, 128), jnp.int32),
            pltpu.VMEM((chg, EMB), F32),
            pltpu.VMEM((chg, EMB), jnp.int32),
            pltpu.SemaphoreType.DMA,
        ],
    )
    def gather_k(g_hbm, hv_hbm, row_hbm, col_hbm, za_hbm, zbv_hbm,
                 idxr, idxc, bufa, bufb, sem):
        wid = lax.axis_index("s") * 2 + lax.axis_index("c")

        def chunk(k, carry):
            ci = wid + NW * k

            @pl.when(ci < nch)
            def _():
                pltpu.sync_copy(row_hbm.at[pl.ds(ci * seg_g, seg_g)], idxr)
                pltpu.sync_copy(col_hbm.at[pl.ds(ci * seg_g, seg_g)], idxc)
                cps = [pltpu.async_copy(g_hbm.at[idxr.at[j]],
                                        bufa.at[pl.ds(j * 128, 128)], sem)
                       for j in range(seg_g)]
                cps += [pltpu.async_copy(hv_hbm.at[idxc.at[j]],
                                         bufb.at[pl.ds(j * 128, 128)], sem)
                        for j in range(seg_g)]
                for cp in cps:
                    cp.wait()
                pltpu.sync_copy(bufa, za_hbm.at[pl.ds(ci * chg, chg)])
                pltpu.sync_copy(bufb, zbv_hbm.at[pl.ds(ci * chg, chg)])

            return carry

        lax.fori_loop(0, iters, chunk, 0)

    return gather_k


def _make_scatter(npad, e):
    chs = 256
    seg_s = chs // 128
    nch = e // chs
    iters = (nch + NW - 1) // NW
    rows_per_tile = npad // 16
    nzc = rows_per_tile // 128
    mesh = plsc.VectorSubcoreMesh(core_axis_name="c", subcore_axis_name="s")

    @functools.partial(
        pl.kernel, mesh=mesh,
        out_type=jax.ShapeDtypeStruct((2, npad, EMB), F32),
        scratch_types=[
            pltpu.VMEM((seg_s, 128), jnp.int32),
            pltpu.VMEM((chs, EMB), F32),
            pltpu.VMEM_SHARED((npad, EMB), F32),
        ],
    )
    def scatter_k(h2_hbm, row_hbm, zeros_hbm, s_hbm, idxr, buf, acc):
        cid = lax.axis_index("c")
        sid = lax.axis_index("s")
        wid = sid * 2 + cid
        base = sid * rows_per_tile
        for kk in range(nzc):
            pltpu.sync_copy(zeros_hbm, acc.at[pl.ds(base + kk * 128, 128)])
        plsc.subcore_barrier()

        def chunk(k, carry):
            ci = wid + NW * k

            @pl.when(ci < nch)
            def _():
                pltpu.sync_copy(row_hbm.at[pl.ds(ci * seg_s, seg_s)], idxr)
                pltpu.sync_copy(h2_hbm.at[pl.ds(ci * chs, chs)], buf)
                for j in range(seg_s):
                    pltpu.sync_copy(buf.at[pl.ds(j * 128, 128)],
                                    acc.at[idxr.at[j]], add=True)

            return carry

        lax.fori_loop(0, iters, chunk, 0)
        plsc.subcore_barrier()
        for kk in range(nzc):
            sl = pl.ds(base + kk * 128, 128)
            pltpu.sync_copy(acc.at[sl], s_hbm.at[cid, sl])

    return scatter_k


def _make_counts(npad, e):
    nch = e // CH
    iters = (nch + NW - 1) // NW
    rows_per_tile = npad // 16
    nzc = rows_per_tile // 128
    mesh = plsc.VectorSubcoreMesh(core_axis_name="c", subcore_axis_name="s")

    @functools.partial(
        pl.kernel, mesh=mesh,
        out_type=jax.ShapeDtypeStruct((2, npad, EMB), F32),
        scratch_types=[
            pltpu.VMEM((SEG, 128), jnp.int32),
            pltpu.VMEM((128, EMB), F32),
            pltpu.VMEM_SHARED((npad, EMB), F32),
        ],
    )
    def counts_k(row_hbm, ones_hbm, zeros_hbm, c_hbm, idxr, ones_v, acc):
        cid = lax.axis_index("c")
        sid = lax.axis_index("s")
        wid = sid * 2 + cid
        base = sid * rows_per_tile
        pltpu.sync_copy(ones_hbm, ones_v)
        for kk in range(nzc):
            pltpu.sync_copy(zeros_hbm, acc.at[pl.ds(base + kk * 128, 128)])
        plsc.subcore_barrier()

        def chunk(k, carry):
            ci = wid + NW * k

            @pl.when(ci < nch)
            def _():
                pltpu.sync_copy(row_hbm.at[pl.ds(ci * SEG, SEG)], idxr)
                for j in range(SEG):
                    pltpu.sync_copy(ones_v, acc.at[idxr.at[j]], add=True)

            return carry

        lax.fori_loop(0, iters, chunk, 0)
        plsc.subcore_barrier()
        for kk in range(nzc):
            sl = pl.ds(base + kk * 128, 128)
            pltpu.sync_copy(acc.at[sl], c_hbm.at[cid, sl])

    return counts_k


def kernel(x, v, edge_index, edge_rots, edge_attr, ptr, params):
    n, in_dim = x.shape
    e = edge_index.shape[1]
    npad = ((n + BN - 1) // BN) * BN

    row = edge_index[0].astype(jnp.int32)
    col = edge_index[1].astype(jnp.int32)
    row2d = row.reshape(e // 128, 128)
    col2d = col.reshape(e // 128, 128)
    rot16 = jnp.pad(edge_rots.reshape(e, 9), ((0, 0), (0, 7)))
    xp = jnp.pad(x, ((0, npad - n), (0, 0)))
    v16 = jnp.pad(v, ((0, npad - n), (0, 4)))

    ones128 = jnp.ones((128, EMB), F32)
    zeros128 = jnp.zeros((128, EMB), F32)

    rs = lambda a: a.reshape(1, -1)

    grid_n = npad // BN
    node_sp = lambda w: pl.BlockSpec((BN, w), _node_block)
    edge_sp = lambda w: pl.BlockSpec((BE, w), _node_block)
    full_sp = lambda s: pl.BlockSpec(s, _full_block)

    h = pl.pallas_call(
        _prologue_body,
        grid=(grid_n,),
        in_specs=[node_sp(in_dim), full_sp((1, in_dim)), full_sp((1, in_dim)),
                  full_sp((in_dim, EMB)), full_sp((1, EMB))],
        out_specs=node_sp(EMB),
        out_shape=jax.ShapeDtypeStruct((npad, EMB), F32),
    )(xp, rs(params['in_g']), rs(params['in_b']), params['proj_W'].T,
      rs(params['proj_b']))

    counts_k = _make_counts(npad, e)
    cpair = counts_k(row2d, ones128, zeros128)
    c0, c1 = cpair[0], cpair[1]

    gather_k = _make_gather(npad, e)
    scatter_k = _make_scatter(npad, e)

    inv = jnp.array([3 * m + i for i in range(3) for m in range(4)],
                    dtype=jnp.int32)

    for p in params['layers']:
        W1 = p['W1']
        W1xi = W1[:, 0:EMB]
        W1xj = W1[:, EMB:2 * EMB]
        W1vi = W1[:, 2 * EMB:2 * EMB + MD]
        W1vr = W1[:, 2 * EMB + MD:2 * EMB + 2 * MD]
        W1s1 = W1[:, 2 * EMB + 2 * MD:2 * EMB + 3 * MD]
        W1ea = W1[:, 2 * EMB + 3 * MD:]
        wgv = jnp.pad((W1vi + W1s1).T, ((0, 4), (0, 0)))
        wd = jnp.pad((W1vr - W1s1).T[inv], ((0, 4), (0, 0)))
        wea = W1ea.T
        wvh = jnp.pad(p['Wv'][:, :EMB].T, ((0, 0), (0, 4)))
        wvv = jnp.pad(p['Wv'][:, EMB:].T, ((0, 4), (0, 4)))

        vt128 = jnp.pad(
            v16[:, :MD].reshape(npad, 4, 3).transpose(0, 2, 1).reshape(npad, MD),
            ((0, 0), (0, EMB - MD)))

        G, Hv = pl.pallas_call(
            _gh_body,
            grid=(grid_n,),
            in_specs=[node_sp(EMB), node_sp(16), node_sp(EMB),
                      full_sp((EMB, EMB)), full_sp((16, EMB)),
                      full_sp((EMB, EMB))],
            out_specs=(node_sp(EMB), node_sp(EMB)),
            out_shape=(jax.ShapeDtypeStruct((npad, EMB), F32),
                       jax.ShapeDtypeStruct((npad, EMB), jnp.int32)),
        )(h, v16, vt128, W1xi.T, wgv, W1xj.T)

        za, zbv = gather_k(G, Hv, row2d, col2d)

        h2 = pl.pallas_call(
            _mlp_body,
            grid=(e // BE,),
            in_specs=[edge_sp(EMB), edge_sp(EMB), edge_sp(16),
                      edge_sp(16), full_sp((16, EMB)), full_sp((16, EMB)),
                      full_sp((1, EMB)), full_sp((EMB, EMB)),
                      full_sp((1, EMB))],
            out_specs=edge_sp(EMB),
            out_shape=jax.ShapeDtypeStruct((e, EMB), F32),
        )(za, zbv, rot16, edge_attr, wd, wea, rs(p['b1']), p['W2'].T,
          rs(p['b2']))

        spair = scatter_k(h2, row2d, zeros128)
        S0, S1 = spair[0], spair[1]

        h, v16 = pl.pallas_call(
            _node_body,
            grid=(grid_n,),
            in_specs=[node_sp(EMB), node_sp(16), node_sp(EMB), node_sp(EMB),
                      node_sp(EMB), node_sp(EMB), full_sp((EMB, EMB)),
                      full_sp((1, EMB)), full_sp((1, EMB)), full_sp((1, EMB)),
                      full_sp((EMB, 16)), full_sp((16, 16))],
            out_specs=(node_sp(EMB), node_sp(16)),
            out_shape=(jax.ShapeDtypeStruct((npad, EMB), F32),
                       jax.ShapeDtypeStruct((npad, 16), F32)),
        )(h, v16, S0, S1, c0, c1, p['W3'].T, rs(p['b3']), rs(p['ln_g']),
          rs(p['ln_b']), wvh, wvv)

    return h[:n], v16[:n, :MD]


# speedup vs baseline: 6.2189x; 1.5057x over previous
import functools

import numpy as np

import jax
import jax.numpy as jnp
from jax import lax
from jax.experimental import pallas as pl
from jax.experimental.pallas import tpu as pltpu
from jax.experimental.pallas import tpu_sc as plsc

F32 = jnp.float32
EMB = 128
MD = 12
NW = 32
CH = 512
SEG = CH // 128
BN = 1024
BE = 2560


def _node_block(i):
    return (i, 0)


def _full_block(i):
    return (0, 0)


def _prologue_body(xr, gr, br, wtr, pbr, out):
    xx = xr[...]
    mu = jnp.mean(xx, axis=-1, keepdims=True)
    var = jnp.mean((xx - mu) ** 2, axis=-1, keepdims=True)
    xn = (xx - mu) * lax.rsqrt(var + 1e-5) * gr[...] + br[...]
    out[...] = jnp.dot(xn, wtr[...], preferred_element_type=F32) + pbr[...]


def _gh_body(hr, vr, vtr, wghr, wgvr, whr, gout, hvout):
    hh = hr[...]
    vv = vr[...]
    gout[...] = (jnp.dot(hh, wghr[...], preferred_element_type=F32)
                 + jnp.dot(vv, wgvr[...], preferred_element_type=F32))
    hcol = jnp.dot(hh, whr[...], preferred_element_type=F32)
    hu = lax.bitcast_convert_type(hcol.astype(jnp.bfloat16),
                                  jnp.uint16).astype(jnp.uint32)
    vu = lax.bitcast_convert_type(vtr[...].astype(jnp.bfloat16),
                                  jnp.uint16).astype(jnp.uint32)
    hvout[...] = lax.bitcast_convert_type(hu | (vu << 16), jnp.int32)


def _mlp_body(zar, zbvr, rotr, attrr, p0r, p1r, p2r, wd0r, wd1r, wd2r, wear,
              b1r, w2r, b2r, out):
    zu = lax.bitcast_convert_type(zbvr[...], jnp.uint32)
    hcol = lax.bitcast_convert_type((zu & 0xFFFF).astype(jnp.uint16),
                                    jnp.bfloat16).astype(F32)
    V = lax.bitcast_convert_type((zu[:, :16] >> 16).astype(jnp.uint16),
                                 jnp.bfloat16).astype(F32)
    z = zar[...] + hcol + b1r[...]
    Rr = rotr[...]
    for pr, wdr in ((p0r, wd0r), (p1r, wd1r), (p2r, wd2r)):
        Ri = jnp.dot(Rr, pr[...], preferred_element_type=F32)
        z = z + jnp.dot(Ri * V, wdr[...], preferred_element_type=F32)
    z = z + jnp.dot(attrr[...], wear[...], preferred_element_type=F32)
    u = jax.nn.gelu(z)
    h2 = jax.nn.gelu(jnp.dot(u, w2r[...], preferred_element_type=F32) + b2r[...])
    out[...] = h2


def _node_body(hr, vr, s0r, s1r, c0r, c1r, w3r, b3r, gr, br, wvhr, wvvr,
               hout, vout):
    S = s0r[...] + s1r[...]
    c = c0r[:, 0:1] + c1r[:, 0:1]
    cmax = jnp.maximum(c, 1.0)
    ind = jnp.minimum(c, 1.0)
    upd = jnp.dot(S, w3r[...], preferred_element_type=F32) / cmax + b3r[...] * ind
    t = hr[...] + upd
    mu = jnp.mean(t, axis=-1, keepdims=True)
    var = jnp.mean((t - mu) ** 2, axis=-1, keepdims=True)
    hn = (t - mu) * lax.rsqrt(var + 1e-5) * gr[...] + br[...]
    hout[...] = hn
    vout[...] = (vr[...] + jnp.dot(hn, wvhr[...], preferred_element_type=F32)
                 + jnp.dot(vr[...], wvvr[...], preferred_element_type=F32))


def _make_gather(npad, e):
    chg = 256
    seg_g = chg // 128
    nch = e // chg
    iters = (nch + NW - 1) // NW
    mesh = plsc.VectorSubcoreMesh(core_axis_name="c", subcore_axis_name="s")

    @functools.partial(
        pl.kernel, mesh=mesh,
        out_type=(jax.ShapeDtypeStruct((e, EMB), F32),
                  jax.ShapeDtypeStruct((e, EMB), jnp.int32)),
        scratch_types=[
            pltpu.VMEM((seg_g, 128), jnp.int32),
            pltpu.VMEM((seg_g, 128), jnp.int32),
            pltpu.VMEM((chg, EMB), F32),
            pltpu.VMEM((chg, EMB), jnp.int32),
            pltpu.SemaphoreType.DMA,
        ],
    )
    def gather_k(g_hbm, hv_hbm, row_hbm, col_hbm, za_hbm, zbv_hbm,
                 idxr, idxc, bufa, bufb, sem):
        wid = lax.axis_index("s") * 2 + lax.axis_index("c")

        def chunk(k, carry):
            ci = wid + NW * k

            @pl.when(ci < nch)
            def _():
                pltpu.sync_copy(row_hbm.at[pl.ds(ci * seg_g, seg_g)], idxr)
                pltpu.sync_copy(col_hbm.at[pl.ds(ci * seg_g, seg_g)], idxc)
                cps = [pltpu.async_copy(g_hbm.at[idxr.at[j]],
                                        bufa.at[pl.ds(j * 128, 128)], sem)
                       for j in range(seg_g)]
                cps += [pltpu.async_copy(hv_hbm.at[idxc.at[j]],
                                         bufb.at[pl.ds(j * 128, 128)], sem)
                        for j in range(seg_g)]
                for cp in cps:
                    cp.wait()
                pltpu.sync_copy(bufa, za_hbm.at[pl.ds(ci * chg, chg)])
                pltpu.sync_copy(bufb, zbv_hbm.at[pl.ds(ci * chg, chg)])

            return carry

        lax.fori_loop(0, iters, chunk, 0)

    return gather_k


def _make_scatter(npad, e):
    chs = 256
    seg_s = chs // 128
    nch = e // chs
    iters = (nch + NW - 1) // NW
    rows_per_tile = npad // 16
    nzc = rows_per_tile // 128
    mesh = plsc.VectorSubcoreMesh(core_axis_name="c", subcore_axis_name="s")

    @functools.partial(
        pl.kernel, mesh=mesh,
        out_type=jax.ShapeDtypeStruct((2, npad, EMB), F32),
        scratch_types=[
            pltpu.VMEM((seg_s, 128), jnp.int32),
            pltpu.VMEM((chs, EMB), F32),
            pltpu.VMEM_SHARED((npad, EMB), F32),
        ],
    )
    def scatter_k(h2_hbm, row_hbm, zeros_hbm, s_hbm, idxr, buf, acc):
        cid = lax.axis_index("c")
        sid = lax.axis_index("s")
        wid = sid * 2 + cid
        base = sid * rows_per_tile
        for kk in range(nzc):
            pltpu.sync_copy(zeros_hbm, acc.at[pl.ds(base + kk * 128, 128)])
        plsc.subcore_barrier()

        def chunk(k, carry):
            ci = wid + NW * k

            @pl.when(ci < nch)
            def _():
                pltpu.sync_copy(row_hbm.at[pl.ds(ci * seg_s, seg_s)], idxr)
                pltpu.sync_copy(h2_hbm.at[pl.ds(ci * chs, chs)], buf)
                for j in range(seg_s):
                    pltpu.sync_copy(buf.at[pl.ds(j * 128, 128)],
                                    acc.at[idxr.at[j]], add=True)

            return carry

        lax.fori_loop(0, iters, chunk, 0)
        plsc.subcore_barrier()
        for kk in range(nzc):
            sl = pl.ds(base + kk * 128, 128)
            pltpu.sync_copy(acc.at[sl], s_hbm.at[cid, sl])

    return scatter_k


def _make_counts(npad, e):
    nch = e // CH
    iters = (nch + NW - 1) // NW
    rows_per_tile = npad // 16
    nzc = rows_per_tile // 128
    mesh = plsc.VectorSubcoreMesh(core_axis_name="c", subcore_axis_name="s")

    @functools.partial(
        pl.kernel, mesh=mesh,
        out_type=jax.ShapeDtypeStruct((2, npad, EMB), F32),
        scratch_types=[
            pltpu.VMEM((SEG, 128), jnp.int32),
            pltpu.VMEM((128, EMB), F32),
            pltpu.VMEM_SHARED((npad, EMB), F32),
        ],
    )
    def counts_k(row_hbm, ones_hbm, zeros_hbm, c_hbm, idxr, ones_v, acc):
        cid = lax.axis_index("c")
        sid = lax.axis_index("s")
        wid = sid * 2 + cid
        base = sid * rows_per_tile
        pltpu.sync_copy(ones_hbm, ones_v)
        for kk in range(nzc):
            pltpu.sync_copy(zeros_hbm, acc.at[pl.ds(base + kk * 128, 128)])
        plsc.subcore_barrier()

        def chunk(k, carry):
            ci = wid + NW * k

            @pl.when(ci < nch)
            def _():
                pltpu.sync_copy(row_hbm.at[pl.ds(ci * SEG, SEG)], idxr)
                for j in range(SEG):
                    pltpu.sync_copy(ones_v, acc.at[idxr.at[j]], add=True)

            return carry

        lax.fori_loop(0, iters, chunk, 0)
        plsc.subcore_barrier()
        for kk in range(nzc):
            sl = pl.ds(base + kk * 128, 128)
            pltpu.sync_copy(acc.at[sl], c_hbm.at[cid, sl])

    return counts_k


def kernel(x, v, edge_index, edge_rots, edge_attr, ptr, params):
    n, in_dim = x.shape
    e = edge_index.shape[1]
    npad = ((n + BN - 1) // BN) * BN

    row = edge_index[0].astype(jnp.int32)
    col = edge_index[1].astype(jnp.int32)
    row2d = row.reshape(e // 128, 128)
    col2d = col.reshape(e // 128, 128)
    rot9 = edge_rots.reshape(e, 9)
    xp = jnp.pad(x, ((0, npad - n), (0, 0)))
    v16 = jnp.pad(v, ((0, npad - n), (0, 4)))

    ones128 = jnp.ones((128, EMB), F32)
    zeros128 = jnp.zeros((128, EMB), F32)

    rs = lambda a: a.reshape(1, -1)

    grid_n = npad // BN
    node_sp = lambda w: pl.BlockSpec((BN, w), _node_block)
    edge_sp = lambda w: pl.BlockSpec((BE, w), _node_block)
    full_sp = lambda s: pl.BlockSpec(s, _full_block)

    h = pl.pallas_call(
        _prologue_body,
        grid=(grid_n,),
        in_specs=[node_sp(in_dim), full_sp((1, in_dim)), full_sp((1, in_dim)),
                  full_sp((in_dim, EMB)), full_sp((1, EMB))],
        out_specs=node_sp(EMB),
        out_shape=jax.ShapeDtypeStruct((npad, EMB), F32),
    )(xp, rs(params['in_g']), rs(params['in_b']), params['proj_W'].T,
      rs(params['proj_b']))

    counts_k = _make_counts(npad, e)
    cpair = counts_k(row2d, ones128, zeros128)
    c0, c1 = cpair[0], cpair[1]

    gather_k = _make_gather(npad, e)
    scatter_k = _make_scatter(npad, e)

    inv = jnp.array([3 * m + i for i in range(3) for m in range(4)],
                    dtype=jnp.int32)
    pmats = []
    for i in range(3):
        pm = np.zeros((9, 16), np.float32)
        for j in range(3):
            for m in range(4):
                pm[3 * i + j, 4 * j + m] = 1.0
        pmats.append(jnp.asarray(pm))

    for p in params['layers']:
        W1 = p['W1']
        W1xi = W1[:, 0:EMB]
        W1xj = W1[:, EMB:2 * EMB]
        W1vi = W1[:, 2 * EMB:2 * EMB + MD]
        W1vr = W1[:, 2 * EMB + MD:2 * EMB + 2 * MD]
        W1s1 = W1[:, 2 * EMB + 2 * MD:2 * EMB + 3 * MD]
        W1ea = W1[:, 2 * EMB + 3 * MD:]
        wgv = jnp.pad((W1vi + W1s1).T, ((0, 4), (0, 0)))
        wd = jnp.pad((W1vr - W1s1).T[inv], ((0, 4), (0, 0)))
        wds = [jnp.pad(jnp.tile(wd[4 * i:4 * i + 4, :], (3, 1)),
                       ((0, 4), (0, 0))) for i in range(3)]
        wea = W1ea.T
        wvh = jnp.pad(p['Wv'][:, :EMB].T, ((0, 0), (0, 4)))
        wvv = jnp.pad(p['Wv'][:, EMB:].T, ((0, 4), (0, 4)))

        vt128 = jnp.pad(
            v16[:, :MD].reshape(npad, 4, 3).transpose(0, 2, 1).reshape(npad, MD),
            ((0, 0), (0, EMB - MD)))

        G, Hv = pl.pallas_call(
            _gh_body,
            grid=(grid_n,),
            in_specs=[node_sp(EMB), node_sp(16), node_sp(EMB),
                      full_sp((EMB, EMB)), full_sp((16, EMB)),
                      full_sp((EMB, EMB))],
            out_specs=(node_sp(EMB), node_sp(EMB)),
            out_shape=(jax.ShapeDtypeStruct((npad, EMB), F32),
                       jax.ShapeDtypeStruct((npad, EMB), jnp.int32)),
        )(h, v16, vt128, W1xi.T, wgv, W1xj.T)

        za, zbv = gather_k(G, Hv, row2d, col2d)

        h2 = pl.pallas_call(
            _mlp_body,
            grid=(e // BE,),
            in_specs=[edge_sp(EMB), edge_sp(EMB), edge_sp(9), edge_sp(16),
                      full_sp((9, 16)), full_sp((9, 16)), full_sp((9, 16)),
                      full_sp((16, EMB)), full_sp((16, EMB)),
                      full_sp((16, EMB)), full_sp((16, EMB)),
                      full_sp((1, EMB)), full_sp((EMB, EMB)),
                      full_sp((1, EMB))],
            out_specs=edge_sp(EMB),
            out_shape=jax.ShapeDtypeStruct((e, EMB), F32),
        )(za, zbv, rot9, edge_attr, pmats[0], pmats[1], pmats[2],
          wds[0], wds[1], wds[2], wea, rs(p['b1']), p['W2'].T, rs(p['b2']))

        spair = scatter_k(h2, row2d, zeros128)
        S0, S1 = spair[0], spair[1]

        h, v16 = pl.pallas_call(
            _node_body,
            grid=(grid_n,),
            in_specs=[node_sp(EMB), node_sp(16), node_sp(EMB), node_sp(EMB),
                      node_sp(EMB), node_sp(EMB), full_sp((EMB, EMB)),
                      full_sp((1, EMB)), full_sp((1, EMB)), full_sp((1, EMB)),
                      full_sp((EMB, 16)), full_sp((16, 16))],
            out_specs=(node_sp(EMB), node_sp(16)),
            out_shape=(jax.ShapeDtypeStruct((npad, EMB), F32),
                       jax.ShapeDtypeStruct((npad, 16), F32)),
        )(h, v16, S0, S1, c0, c1, p['W3'].T, rs(p['b3']), rs(p['ln_g']),
          rs(p['ln_b']), wvh, wvv)

    return h[:n], v16[:n, :MD]


# speedup vs baseline: 6.5646x; 1.0556x over previous
import functools

import numpy as np

import jax
import jax.numpy as jnp
from jax import lax
from jax.experimental import pallas as pl
from jax.experimental.pallas import tpu as pltpu
from jax.experimental.pallas import tpu_sc as plsc

F32 = jnp.float32
EMB = 128
MD = 12
NW = 32
CH = 512
SEG = CH // 128
BN = 1024
BE = 6400


def _node_block(i):
    return (i, 0)


def _full_block(i):
    return (0, 0)


def _prologue_body(xr, gr, br, wtr, pbr, out):
    xx = xr[...]
    mu = jnp.mean(xx, axis=-1, keepdims=True)
    var = jnp.mean((xx - mu) ** 2, axis=-1, keepdims=True)
    xn = (xx - mu) * lax.rsqrt(var + 1e-5) * gr[...] + br[...]
    out[...] = jnp.dot(xn, wtr[...], preferred_element_type=F32) + pbr[...]


def _gh_body(hr, vr, vtr, wghr, wgvr, whr, gout, hvout):
    hh = hr[...]
    vv = vr[...]
    gout[...] = (jnp.dot(hh, wghr[...], preferred_element_type=F32)
                 + jnp.dot(vv, wgvr[...], preferred_element_type=F32))
    hcol = jnp.dot(hh, whr[...], preferred_element_type=F32)
    hu = lax.bitcast_convert_type(hcol.astype(jnp.bfloat16),
                                  jnp.uint16).astype(jnp.uint32)
    vu = lax.bitcast_convert_type(vtr[...].astype(jnp.bfloat16),
                                  jnp.uint16).astype(jnp.uint32)
    hvout[...] = lax.bitcast_convert_type(hu | (vu << 16), jnp.int32)


def _mlp_body(zar, zbvr, rotr, attrr, p0r, p1r, p2r, wd0r, wd1r, wd2r, wear,
              b1r, w2r, b2r, out):
    zu = lax.bitcast_convert_type(zbvr[...], jnp.uint32)
    hcol = lax.bitcast_convert_type((zu & 0xFFFF).astype(jnp.uint16),
                                    jnp.bfloat16).astype(F32)
    V = lax.bitcast_convert_type((zu[:, :16] >> 16).astype(jnp.uint16),
                                 jnp.bfloat16).astype(F32)
    z = zar[...] + hcol + b1r[...]
    Rr = rotr[...]
    for pr, wdr in ((p0r, wd0r), (p1r, wd1r), (p2r, wd2r)):
        Ri = jnp.dot(Rr, pr[...], preferred_element_type=F32)
        z = z + jnp.dot(Ri * V, wdr[...], preferred_element_type=F32)
    z = z + jnp.dot(attrr[...], wear[...], preferred_element_type=F32)
    u = jax.nn.gelu(z).astype(jnp.bfloat16)
    h2 = jax.nn.gelu(jnp.dot(u, w2r[...], preferred_element_type=F32) + b2r[...])
    out[...] = h2


def _node_body(hr, vr, s0r, s1r, c0r, c1r, w3r, b3r, gr, br, wvhr, wvvr,
               hout, vout):
    S = s0r[...] + s1r[...]
    c = c0r[:, 0:1] + c1r[:, 0:1]
    cmax = jnp.maximum(c, 1.0)
    ind = jnp.minimum(c, 1.0)
    upd = jnp.dot(S, w3r[...], preferred_element_type=F32) / cmax + b3r[...] * ind
    t = hr[...] + upd
    mu = jnp.mean(t, axis=-1, keepdims=True)
    var = jnp.mean((t - mu) ** 2, axis=-1, keepdims=True)
    hn = (t - mu) * lax.rsqrt(var + 1e-5) * gr[...] + br[...]
    hout[...] = hn
    vout[...] = (vr[...] + jnp.dot(hn, wvhr[...], preferred_element_type=F32)
                 + jnp.dot(vr[...], wvvr[...], preferred_element_type=F32))


def _make_gather(npad, e):
    chg = 256
    seg_g = chg // 128
    nch = e // chg
    iters = (nch + NW - 1) // NW
    mesh = plsc.VectorSubcoreMesh(core_axis_name="c", subcore_axis_name="s")

    @functools.partial(
        pl.kernel, mesh=mesh,
        out_type=(jax.ShapeDtypeStruct((e, EMB), F32),
                  jax.ShapeDtypeStruct((e, EMB), jnp.int32)),
        scratch_types=[
            pltpu.VMEM((seg_g, 128), jnp.int32),
            pltpu.VMEM((seg_g, 128), jnp.int32),
            pltpu.VMEM((chg, EMB), F32),
            pltpu.VMEM((chg, EMB), jnp.int32),
            pltpu.SemaphoreType.DMA,
        ],
    )
    def gather_k(g_hbm, hv_hbm, row_hbm, col_hbm, za_hbm, zbv_hbm,
                 idxr, idxc, bufa, bufb, sem):
        wid = lax.axis_index("s") * 2 + lax.axis_index("c")

        def chunk(k, carry):
            ci = wid + NW * k

            @pl.when(ci < nch)
            def _():
                pltpu.sync_copy(row_hbm.at[pl.ds(ci * seg_g, seg_g)], idxr)
                pltpu.sync_copy(col_hbm.at[pl.ds(ci * seg_g, seg_g)], idxc)
                cps = [pltpu.async_copy(g_hbm.at[idxr.at[j]],
                                        bufa.at[pl.ds(j * 128, 128)], sem)
                       for j in range(seg_g)]
                cps += [pltpu.async_copy(hv_hbm.at[idxc.at[j]],
                                         bufb.at[pl.ds(j * 128, 128)], sem)
                        for j in range(seg_g)]
                for cp in cps:
                    cp.wait()
                pltpu.sync_copy(bufa, za_hbm.at[pl.ds(ci * chg, chg)])
                pltpu.sync_copy(bufb, zbv_hbm.at[pl.ds(ci * chg, chg)])

            return carry

        lax.fori_loop(0, iters, chunk, 0)

    return gather_k


def _make_scatter(npad, e):
    chs = 256
    seg_s = chs // 128
    nch = e // chs
    iters = (nch + NW - 1) // NW
    rows_per_tile = npad // 16
    nzc = rows_per_tile // 128
    mesh = plsc.VectorSubcoreMesh(core_axis_name="c", subcore_axis_name="s")

    @functools.partial(
        pl.kernel, mesh=mesh,
        out_type=jax.ShapeDtypeStruct((2, npad, EMB), F32),
        scratch_types=[
            pltpu.VMEM((seg_s, 128), jnp.int32),
            pltpu.VMEM((chs, EMB), F32),
            pltpu.VMEM_SHARED((npad, EMB), F32),
        ],
    )
    def scatter_k(h2_hbm, row_hbm, zeros_hbm, s_hbm, idxr, buf, acc):
        cid = lax.axis_index("c")
        sid = lax.axis_index("s")
        wid = sid * 2 + cid
        base = sid * rows_per_tile
        for kk in range(nzc):
            pltpu.sync_copy(zeros_hbm, acc.at[pl.ds(base + kk * 128, 128)])
        plsc.subcore_barrier()

        def chunk(k, carry):
            ci = wid + NW * k

            @pl.when(ci < nch)
            def _():
                pltpu.sync_copy(row_hbm.at[pl.ds(ci * seg_s, seg_s)], idxr)
                pltpu.sync_copy(h2_hbm.at[pl.ds(ci * chs, chs)], buf)
                for j in range(seg_s):
                    pltpu.sync_copy(buf.at[pl.ds(j * 128, 128)],
                                    acc.at[idxr.at[j]], add=True)

            return carry

        lax.fori_loop(0, iters, chunk, 0)
        plsc.subcore_barrier()
        for kk in range(nzc):
            sl = pl.ds(base + kk * 128, 128)
            pltpu.sync_copy(acc.at[sl], s_hbm.at[cid, sl])

    return scatter_k


def _make_counts(npad, e):
    nch = e // CH
    iters = (nch + NW - 1) // NW
    rows_per_tile = npad // 16
    nzc = rows_per_tile // 128
    mesh = plsc.VectorSubcoreMesh(core_axis_name="c", subcore_axis_name="s")

    @functools.partial(
        pl.kernel, mesh=mesh,
        out_type=jax.ShapeDtypeStruct((2, npad, EMB), F32),
        scratch_types=[
            pltpu.VMEM((SEG, 128), jnp.int32),
            pltpu.VMEM((128, EMB), F32),
            pltpu.VMEM_SHARED((npad, EMB), F32),
        ],
    )
    def counts_k(row_hbm, ones_hbm, zeros_hbm, c_hbm, idxr, ones_v, acc):
        cid = lax.axis_index("c")
        sid = lax.axis_index("s")
        wid = sid * 2 + cid
        base = sid * rows_per_tile
        pltpu.sync_copy(ones_hbm, ones_v)
        for kk in range(nzc):
            pltpu.sync_copy(zeros_hbm, acc.at[pl.ds(base + kk * 128, 128)])
        plsc.subcore_barrier()

        def chunk(k, carry):
            ci = wid + NW * k

            @pl.when(ci < nch)
            def _():
                pltpu.sync_copy(row_hbm.at[pl.ds(ci * SEG, SEG)], idxr)
                for j in range(SEG):
                    pltpu.sync_copy(ones_v, acc.at[idxr.at[j]], add=True)

            return carry

        lax.fori_loop(0, iters, chunk, 0)
        plsc.subcore_barrier()
        for kk in range(nzc):
            sl = pl.ds(base + kk * 128, 128)
            pltpu.sync_copy(acc.at[sl], c_hbm.at[cid, sl])

    return counts_k


def kernel(x, v, edge_index, edge_rots, edge_attr, ptr, params):
    n, in_dim = x.shape
    e = edge_index.shape[1]
    npad = ((n + BN - 1) // BN) * BN

    row = edge_index[0].astype(jnp.int32)
    col = edge_index[1].astype(jnp.int32)
    row2d = row.reshape(e // 128, 128)
    col2d = col.reshape(e // 128, 128)
    rot9 = edge_rots.reshape(e, 9)
    xp = jnp.pad(x, ((0, npad - n), (0, 0)))
    v16 = jnp.pad(v, ((0, npad - n), (0, 4)))

    ones128 = jnp.ones((128, EMB), F32)
    zeros128 = jnp.zeros((128, EMB), F32)

    rs = lambda a: a.reshape(1, -1)

    grid_n = npad // BN
    node_sp = lambda w: pl.BlockSpec((BN, w), _node_block)
    edge_sp = lambda w: pl.BlockSpec((BE, w), _node_block)
    full_sp = lambda s: pl.BlockSpec(s, _full_block)

    h = pl.pallas_call(
        _prologue_body,
        grid=(grid_n,),
        in_specs=[node_sp(in_dim), full_sp((1, in_dim)), full_sp((1, in_dim)),
                  full_sp((in_dim, EMB)), full_sp((1, EMB))],
        out_specs=node_sp(EMB),
        out_shape=jax.ShapeDtypeStruct((npad, EMB), F32),
    )(xp, rs(params['in_g']), rs(params['in_b']), params['proj_W'].T,
      rs(params['proj_b']))

    counts_k = _make_counts(npad, e)
    cpair = counts_k(row2d, ones128, zeros128)
    c0, c1 = cpair[0], cpair[1]

    gather_k = _make_gather(npad, e)
    scatter_k = _make_scatter(npad, e)

    inv = jnp.array([3 * m + i for i in range(3) for m in range(4)],
                    dtype=jnp.int32)
    pmats = []
    for i in range(3):
        pm = np.zeros((9, 16), np.float32)
        for j in range(3):
            for m in range(4):
                pm[3 * i + j, 4 * j + m] = 1.0
        pmats.append(jnp.asarray(pm))

    for p in params['layers']:
        W1 = p['W1']
        W1xi = W1[:, 0:EMB]
        W1xj = W1[:, EMB:2 * EMB]
        W1vi = W1[:, 2 * EMB:2 * EMB + MD]
        W1vr = W1[:, 2 * EMB + MD:2 * EMB + 2 * MD]
        W1s1 = W1[:, 2 * EMB + 2 * MD:2 * EMB + 3 * MD]
        W1ea = W1[:, 2 * EMB + 3 * MD:]
        wgv = jnp.pad((W1vi + W1s1).T, ((0, 4), (0, 0)))
        wd = jnp.pad((W1vr - W1s1).T[inv], ((0, 4), (0, 0)))
        wds = [jnp.pad(jnp.tile(wd[4 * i:4 * i + 4, :], (3, 1)),
                       ((0, 4), (0, 0))) for i in range(3)]
        wea = W1ea.T
        wvh = jnp.pad(p['Wv'][:, :EMB].T, ((0, 0), (0, 4)))
        wvv = jnp.pad(p['Wv'][:, EMB:].T, ((0, 4), (0, 4)))

        vt128 = jnp.pad(
            v16[:, :MD].reshape(npad, 4, 3).transpose(0, 2, 1).reshape(npad, MD),
            ((0, 0), (0, EMB - MD)))

        G, Hv = pl.pallas_call(
            _gh_body,
            grid=(grid_n,),
            in_specs=[node_sp(EMB), node_sp(16), node_sp(EMB),
                      full_sp((EMB, EMB)), full_sp((16, EMB)),
                      full_sp((EMB, EMB))],
            out_specs=(node_sp(EMB), node_sp(EMB)),
            out_shape=(jax.ShapeDtypeStruct((npad, EMB), F32),
                       jax.ShapeDtypeStruct((npad, EMB), jnp.int32)),
        )(h, v16, vt128, W1xi.T, wgv, W1xj.T)

        za, zbv = gather_k(G, Hv, row2d, col2d)

        h2 = pl.pallas_call(
            _mlp_body,
            grid=(e // BE,),
            in_specs=[edge_sp(EMB), edge_sp(EMB), edge_sp(9), edge_sp(16),
                      full_sp((9, 16)), full_sp((9, 16)), full_sp((9, 16)),
                      full_sp((16, EMB)), full_sp((16, EMB)),
                      full_sp((16, EMB)), full_sp((16, EMB)),
                      full_sp((1, EMB)), full_sp((EMB, EMB)),
                      full_sp((1, EMB))],
            out_specs=edge_sp(EMB),
            out_shape=jax.ShapeDtypeStruct((e, EMB), F32),
        )(za, zbv, rot9, edge_attr, pmats[0], pmats[1], pmats[2],
          wds[0], wds[1], wds[2], wea, rs(p['b1']),
          p['W2'].T.astype(jnp.bfloat16), rs(p['b2']))

        spair = scatter_k(h2, row2d, zeros128)
        S0, S1 = spair[0], spair[1]

        h, v16 = pl.pallas_call(
            _node_body,
            grid=(grid_n,),
            in_specs=[node_sp(EMB), node_sp(16), node_sp(EMB), node_sp(EMB),
                      node_sp(EMB), node_sp(EMB), full_sp((EMB, EMB)),
                      full_sp((1, EMB)), full_sp((1, EMB)), full_sp((1, EMB)),
                      full_sp((EMB, 16)), full_sp((16, 16))],
            out_specs=(node_sp(EMB), node_sp(16)),
            out_shape=(jax.ShapeDtypeStruct((npad, EMB), F32),
                       jax.ShapeDtypeStruct((npad, 16), F32)),
        )(h, v16, S0, S1, c0, c1, p['W3'].T, rs(p['b3']), rs(p['ln_g']),
          rs(p['ln_b']), wvh, wvv)

    return h[:n], v16[:n, :MD]


# speedup vs baseline: 6.7023x; 1.0210x over previous
import functools

import numpy as np

import jax
import jax.numpy as jnp
from jax import lax
from jax.experimental import pallas as pl
from jax.experimental.pallas import tpu as pltpu
from jax.experimental.pallas import tpu_sc as plsc

F32 = jnp.float32
EMB = 128
MD = 12
NW = 32
CH = 512
SEG = CH // 128
BN = 1024
BE = 6400


def _node_block(i):
    return (i, 0)


def _full_block(i):
    return (0, 0)


def _prologue_body(xr, gr, br, wtr, pbr, out):
    xx = xr[...]
    mu = jnp.mean(xx, axis=-1, keepdims=True)
    var = jnp.mean((xx - mu) ** 2, axis=-1, keepdims=True)
    xn = (xx - mu) * lax.rsqrt(var + 1e-5) * gr[...] + br[...]
    out[...] = jnp.dot(xn, wtr[...], preferred_element_type=F32) + pbr[...]


def _gh_body(hr, vr, vtr, wghr, wgvr, whr, gout, hvout):
    hh = hr[...]
    vv = vr[...]
    gout[...] = (jnp.dot(hh, wghr[...], preferred_element_type=F32)
                 + jnp.dot(vv, wgvr[...], preferred_element_type=F32))
    hcol = jnp.dot(hh, whr[...], preferred_element_type=F32)
    hu = lax.bitcast_convert_type(hcol.astype(jnp.bfloat16),
                                  jnp.uint16).astype(jnp.uint32)
    vu = lax.bitcast_convert_type(vtr[...].astype(jnp.bfloat16),
                                  jnp.uint16).astype(jnp.uint32)
    hvout[...] = lax.bitcast_convert_type(hu | (vu << 16), jnp.int32)


def _mlp_body(zar, zbvr, rotr, attrr, p0r, p1r, p2r, wd0r, wd1r, wd2r, wear,
              b1r, w2r, b2r, out):
    zu = lax.bitcast_convert_type(zbvr[...], jnp.uint32)
    hcol = lax.bitcast_convert_type((zu & 0xFFFF).astype(jnp.uint16),
                                    jnp.bfloat16).astype(F32)
    V = lax.bitcast_convert_type((zu[:, :16] >> 16).astype(jnp.uint16),
                                 jnp.bfloat16).astype(F32)
    z = zar[...] + hcol + b1r[...]
    Rr = rotr[...]
    for pr, wdr in ((p0r, wd0r), (p1r, wd1r), (p2r, wd2r)):
        Ri = jnp.dot(Rr, pr[...], preferred_element_type=F32)
        z = z + jnp.dot(Ri * V, wdr[...], preferred_element_type=F32)
    z = z + jnp.dot(attrr[...], wear[...], preferred_element_type=F32)
    u = jax.nn.gelu(z).astype(jnp.bfloat16)
    h2 = jax.nn.gelu(jnp.dot(u, w2r[...], preferred_element_type=F32) + b2r[...])
    out[...] = h2


def _node_body(hr, vr, s0r, s1r, c0r, c1r, w3r, b3r, gr, br, wvhr, wvvr,
               hout, vout):
    S = s0r[...] + s1r[...]
    c = c0r[:, 0:1] + c1r[:, 0:1]
    cmax = jnp.maximum(c, 1.0)
    ind = jnp.minimum(c, 1.0)
    upd = jnp.dot(S, w3r[...], preferred_element_type=F32) / cmax + b3r[...] * ind
    t = hr[...] + upd
    mu = jnp.mean(t, axis=-1, keepdims=True)
    var = jnp.mean((t - mu) ** 2, axis=-1, keepdims=True)
    hn = (t - mu) * lax.rsqrt(var + 1e-5) * gr[...] + br[...]
    hout[...] = hn
    vout[...] = (vr[...] + jnp.dot(hn, wvhr[...], preferred_element_type=F32)
                 + jnp.dot(vr[...], wvvr[...], preferred_element_type=F32))


def _make_gather(npad, e_half, ch_off):
    chg = 256
    seg_g = chg // 128
    nch = e_half // chg
    iters = (nch + NW - 1) // NW
    mesh = plsc.VectorSubcoreMesh(core_axis_name="c", subcore_axis_name="s")

    @functools.partial(
        pl.kernel, mesh=mesh,
        out_type=(jax.ShapeDtypeStruct((e_half, EMB), F32),
                  jax.ShapeDtypeStruct((e_half, EMB), jnp.int32)),
        scratch_types=[
            pltpu.VMEM((seg_g, 128), jnp.int32),
            pltpu.VMEM((seg_g, 128), jnp.int32),
            pltpu.VMEM((chg, EMB), F32),
            pltpu.VMEM((chg, EMB), jnp.int32),
            pltpu.SemaphoreType.DMA,
        ],
    )
    def gather_k(g_hbm, hv_hbm, row_hbm, col_hbm, za_hbm, zbv_hbm,
                 idxr, idxc, bufa, bufb, sem):
        wid = lax.axis_index("s") * 2 + lax.axis_index("c")

        def chunk(k, carry):
            ci = wid + NW * k

            @pl.when(ci < nch)
            def _():
                cj = ci + ch_off
                pltpu.sync_copy(row_hbm.at[pl.ds(cj * seg_g, seg_g)], idxr)
                pltpu.sync_copy(col_hbm.at[pl.ds(cj * seg_g, seg_g)], idxc)
                cps = [pltpu.async_copy(g_hbm.at[idxr.at[j]],
                                        bufa.at[pl.ds(j * 128, 128)], sem)
                       for j in range(seg_g)]
                cps += [pltpu.async_copy(hv_hbm.at[idxc.at[j]],
                                         bufb.at[pl.ds(j * 128, 128)], sem)
                        for j in range(seg_g)]
                for cp in cps:
                    cp.wait()
                pltpu.sync_copy(bufa, za_hbm.at[pl.ds(ci * chg, chg)])
                pltpu.sync_copy(bufb, zbv_hbm.at[pl.ds(ci * chg, chg)])

            return carry

        lax.fori_loop(0, iters, chunk, 0)

    return gather_k


def _make_scatter(npad, e):
    chs = 256
    seg_s = chs // 128
    nch = e // chs
    iters = (nch + NW - 1) // NW
    rows_per_tile = npad // 16
    nzc = rows_per_tile // 128
    mesh = plsc.VectorSubcoreMesh(core_axis_name="c", subcore_axis_name="s")

    @functools.partial(
        pl.kernel, mesh=mesh,
        out_type=jax.ShapeDtypeStruct((2, npad, EMB), F32),
        scratch_types=[
            pltpu.VMEM((seg_s, 128), jnp.int32),
            pltpu.VMEM((chs, EMB), F32),
            pltpu.VMEM_SHARED((npad, EMB), F32),
        ],
    )
    def scatter_k(h2a_hbm, h2b_hbm, row_hbm, zeros_hbm, s_hbm, idxr, buf, acc):
        cid = lax.axis_index("c")
        sid = lax.axis_index("s")
        wid = sid * 2 + cid
        base = sid * rows_per_tile
        half = nch // 2
        for kk in range(nzc):
            pltpu.sync_copy(zeros_hbm, acc.at[pl.ds(base + kk * 128, 128)])
        plsc.subcore_barrier()

        def make_chunk(h2_hbm, ci0):
            def chunk(k, carry):
                ci = ci0 + wid + NW * k

                @pl.when(ci < ci0 + half)
                def _():
                    pltpu.sync_copy(row_hbm.at[pl.ds(ci * seg_s, seg_s)], idxr)
                    pltpu.sync_copy(
                        h2_hbm.at[pl.ds((ci - ci0) * chs, chs)], buf)
                    for j in range(seg_s):
                        pltpu.sync_copy(buf.at[pl.ds(j * 128, 128)],
                                        acc.at[idxr.at[j]], add=True)

                return carry
            return chunk

        lax.fori_loop(0, (half + NW - 1) // NW, make_chunk(h2a_hbm, 0), 0)
        lax.fori_loop(0, (half + NW - 1) // NW, make_chunk(h2b_hbm, half), 0)
        plsc.subcore_barrier()
        for kk in range(nzc):
            sl = pl.ds(base + kk * 128, 128)
            pltpu.sync_copy(acc.at[sl], s_hbm.at[cid, sl])

    return scatter_k


def _make_counts(npad, e):
    nch = e // CH
    iters = (nch + NW - 1) // NW
    rows_per_tile = npad // 16
    nzc = rows_per_tile // 128
    mesh = plsc.VectorSubcoreMesh(core_axis_name="c", subcore_axis_name="s")

    @functools.partial(
        pl.kernel, mesh=mesh,
        out_type=jax.ShapeDtypeStruct((2, npad, EMB), F32),
        scratch_types=[
            pltpu.VMEM((SEG, 128), jnp.int32),
            pltpu.VMEM((128, EMB), F32),
            pltpu.VMEM_SHARED((npad, EMB), F32),
        ],
    )
    def counts_k(row_hbm, ones_hbm, zeros_hbm, c_hbm, idxr, ones_v, acc):
        cid = lax.axis_index("c")
        sid = lax.axis_index("s")
        wid = sid * 2 + cid
        base = sid * rows_per_tile
        pltpu.sync_copy(ones_hbm, ones_v)
        for kk in range(nzc):
            pltpu.sync_copy(zeros_hbm, acc.at[pl.ds(base + kk * 128, 128)])
        plsc.subcore_barrier()

        def chunk(k, carry):
            ci = wid + NW * k

            @pl.when(ci < nch)
            def _():
                pltpu.sync_copy(row_hbm.at[pl.ds(ci * SEG, SEG)], idxr)
                for j in range(SEG):
                    pltpu.sync_copy(ones_v, acc.at[idxr.at[j]], add=True)

            return carry

        lax.fori_loop(0, iters, chunk, 0)
        plsc.subcore_barrier()
        for kk in range(nzc):
            sl = pl.ds(base + kk * 128, 128)
            pltpu.sync_copy(acc.at[sl], c_hbm.at[cid, sl])

    return counts_k


def kernel(x, v, edge_index, edge_rots, edge_attr, ptr, params):
    n, in_dim = x.shape
    e = edge_index.shape[1]
    npad = ((n + BN - 1) // BN) * BN

    row = edge_index[0].astype(jnp.int32)
    col = edge_index[1].astype(jnp.int32)
    row2d = row.reshape(e // 128, 128)
    col2d = col.reshape(e // 128, 128)
    rot9 = edge_rots.reshape(e, 9)
    xp = jnp.pad(x, ((0, npad - n), (0, 0)))
    v16 = jnp.pad(v, ((0, npad - n), (0, 4)))

    ones128 = jnp.ones((128, EMB), F32)
    zeros128 = jnp.zeros((128, EMB), F32)

    rs = lambda a: a.reshape(1, -1)

    grid_n = npad // BN
    node_sp = lambda w: pl.BlockSpec((BN, w), _node_block)
    edge_sp = lambda w: pl.BlockSpec((BE, w), _node_block)
    full_sp = lambda s: pl.BlockSpec(s, _full_block)

    h = pl.pallas_call(
        _prologue_body,
        grid=(grid_n,),
        in_specs=[node_sp(in_dim), full_sp((1, in_dim)), full_sp((1, in_dim)),
                  full_sp((in_dim, EMB)), full_sp((1, EMB))],
        out_specs=node_sp(EMB),
        out_shape=jax.ShapeDtypeStruct((npad, EMB), F32),
    )(xp, rs(params['in_g']), rs(params['in_b']), params['proj_W'].T,
      rs(params['proj_b']))

    counts_k = _make_counts(npad, e)
    cpair = counts_k(row2d, ones128, zeros128)
    c0, c1 = cpair[0], cpair[1]

    eh = e // 2
    gather_a = _make_gather(npad, eh, 0)
    gather_b = _make_gather(npad, eh, eh // 256)
    scatter_k = _make_scatter(npad, e)

    inv = jnp.array([3 * m + i for i in range(3) for m in range(4)],
                    dtype=jnp.int32)
    pmats = []
    for i in range(3):
        pm = np.zeros((9, 16), np.float32)
        for j in range(3):
            for m in range(4):
                pm[3 * i + j, 4 * j + m] = 1.0
        pmats.append(jnp.asarray(pm))

    for p in params['layers']:
        W1 = p['W1']
        W1xi = W1[:, 0:EMB]
        W1xj = W1[:, EMB:2 * EMB]
        W1vi = W1[:, 2 * EMB:2 * EMB + MD]
        W1vr = W1[:, 2 * EMB + MD:2 * EMB + 2 * MD]
        W1s1 = W1[:, 2 * EMB + 2 * MD:2 * EMB + 3 * MD]
        W1ea = W1[:, 2 * EMB + 3 * MD:]
        wgv = jnp.pad((W1vi + W1s1).T, ((0, 4), (0, 0)))
        wd = jnp.pad((W1vr - W1s1).T[inv], ((0, 4), (0, 0)))
        wds = [jnp.pad(jnp.tile(wd[4 * i:4 * i + 4, :], (3, 1)),
                       ((0, 4), (0, 0))) for i in range(3)]
        wea = W1ea.T
        wvh = jnp.pad(p['Wv'][:, :EMB].T, ((0, 0), (0, 4)))
        wvv = jnp.pad(p['Wv'][:, EMB:].T, ((0, 4), (0, 4)))

        vt128 = jnp.pad(
            v16[:, :MD].reshape(npad, 4, 3).transpose(0, 2, 1).reshape(npad, MD),
            ((0, 0), (0, EMB - MD)))

        G, Hv = pl.pallas_call(
            _gh_body,
            grid=(grid_n,),
            in_specs=[node_sp(EMB), node_sp(16), node_sp(EMB),
                      full_sp((EMB, EMB)), full_sp((16, EMB)),
                      full_sp((EMB, EMB))],
            out_specs=(node_sp(EMB), node_sp(EMB)),
            out_shape=(jax.ShapeDtypeStruct((npad, EMB), F32),
                       jax.ShapeDtypeStruct((npad, EMB), jnp.int32)),
        )(h, v16, vt128, W1xi.T, wgv, W1xj.T)

        halves = []
        for gk, off in ((gather_a, 0), (gather_b, eh // BE)):
            za, zbv = gk(G, Hv, row2d, col2d)
            off_sp = pl.BlockSpec((BE, 9), lambda i, _o=off: (i + _o, 0))
            off_sp16 = pl.BlockSpec((BE, 16), lambda i, _o=off: (i + _o, 0))
            h2h = pl.pallas_call(
                _mlp_body,
                grid=(eh // BE,),
                in_specs=[edge_sp(EMB), edge_sp(EMB), off_sp, off_sp16,
                          full_sp((9, 16)), full_sp((9, 16)), full_sp((9, 16)),
                          full_sp((16, EMB)), full_sp((16, EMB)),
                          full_sp((16, EMB)), full_sp((16, EMB)),
                          full_sp((1, EMB)), full_sp((EMB, EMB)),
                          full_sp((1, EMB))],
                out_specs=edge_sp(EMB),
                out_shape=jax.ShapeDtypeStruct((eh, EMB), F32),
            )(za, zbv, rot9, edge_attr, pmats[0], pmats[1], pmats[2],
              wds[0], wds[1], wds[2], wea, rs(p['b1']),
              p['W2'].T.astype(jnp.bfloat16), rs(p['b2']))
            halves.append(h2h)

        spair = scatter_k(halves[0], halves[1], row2d, zeros128)
        S0, S1 = spair[0], spair[1]

        h, v16 = pl.pallas_call(
            _node_body,
            grid=(grid_n,),
            in_specs=[node_sp(EMB), node_sp(16), node_sp(EMB), node_sp(EMB),
                      node_sp(EMB), node_sp(EMB), full_sp((EMB, EMB)),
                      full_sp((1, EMB)), full_sp((1, EMB)), full_sp((1, EMB)),
                      full_sp((EMB, 16)), full_sp((16, 16))],
            out_specs=(node_sp(EMB), node_sp(16)),
            out_shape=(jax.ShapeDtypeStruct((npad, EMB), F32),
                       jax.ShapeDtypeStruct((npad, 16), F32)),
        )(h, v16, S0, S1, c0, c1, p['W3'].T, rs(p['b3']), rs(p['ln_g']),
          rs(p['ln_b']), wvh, wvv)

    return h[:n], v16[:n, :MD]
